# baseline (speedup 1.0000x reference)
import jax
import jax.numpy as jnp
from jax.experimental import pallas as pl
from jax.experimental.pallas import tpu as pltpu

_BM = 400
_SPLIT = 2
_BS = _BM // _SPLIT


def _agg_kernel(feature_ref, adj_ref, w_ref, b_ref, out_ref, support_ref):
    i = pl.program_id(0)
    j = pl.program_id(1)

    @pl.when((i == 0) & (j == 0))
    def _():
        support_ref[...] = jnp.dot(
            feature_ref[...], w_ref[...],
            preferred_element_type=jnp.float32) + b_ref[...]

    a = adj_ref[pl.ds(j * _BS, _BS), :]
    h = jnp.dot(a, support_ref[...], preferred_element_type=jnp.float32)
    out_ref[...] = jnp.tanh(h)


def kernel(feature, adj, W, b):
    n, d = feature.shape
    b2 = b.reshape(1, d)
    return pl.pallas_call(
        _agg_kernel,
        grid=(n // _BM, _SPLIT),
        in_specs=[
            pl.BlockSpec((n, d), lambda i, j: (0, 0)),
            pl.BlockSpec((_BM, n), lambda i, j: (i, 0)),
            pl.BlockSpec((d, d), lambda i, j: (0, 0)),
            pl.BlockSpec((1, d), lambda i, j: (0, 0)),
        ],
        out_specs=pl.BlockSpec((_BS, d), lambda i, j: (i * _SPLIT + j, 0)),
        out_shape=jax.ShapeDtypeStruct((n, d), jnp.float32),
        scratch_shapes=[pltpu.VMEM((n, d), jnp.float32)],
        compiler_params=pltpu.CompilerParams(
            dimension_semantics=("arbitrary", "arbitrary"),
        ),
    )(feature, adj, W, b2)


# speedup vs baseline: 1.4347x; 1.4347x over previous
import jax
import jax.numpy as jnp
from jax.experimental import pallas as pl
from jax.experimental.pallas import tpu as pltpu

_BM = 400


def _agg_kernel(feature_ref, adj_ref, w_ref, b_ref, out_ref, support_ref):
    @pl.when(pl.program_id(0) == 0)
    def _():
        support_ref[...] = jnp.dot(
            feature_ref[...], w_ref[...],
            preferred_element_type=jnp.float32) + b_ref[...]

    h = jnp.dot(adj_ref[...], support_ref[...],
                preferred_element_type=jnp.float32)
    out_ref[...] = jnp.tanh(h)


def kernel(feature, adj, W, b):
    n, d = feature.shape
    b2 = b.reshape(1, d)
    return pl.pallas_call(
        _agg_kernel,
        grid=(n // _BM,),
        in_specs=[
            pl.BlockSpec((n, d), lambda i: (0, 0)),
            pl.BlockSpec((_BM, n), lambda i: (i, 0)),
            pl.BlockSpec((d, d), lambda i: (0, 0)),
            pl.BlockSpec((1, d), lambda i: (0, 0)),
        ],
        out_specs=pl.BlockSpec((_BM, d), lambda i: (i, 0)),
        out_shape=jax.ShapeDtypeStruct((n, d), jnp.float32),
        scratch_shapes=[pltpu.VMEM((n, d), jnp.float32)],
        compiler_params=pltpu.CompilerParams(
            dimension_semantics=("arbitrary",),
        ),
    )(feature, adj, W, b2)
